# Initial kernel scaffold; baseline (speedup 1.0000x reference)
import jax
import jax.numpy as jnp
from jax.experimental import pallas as pl


def kernel(inputs, table):
    raise NotImplementedError("write your pallas kernel here")


# speedup vs baseline: 8.9398x; 8.9398x over previous
import jax
import jax.numpy as jnp
from jax.experimental import pallas as pl
from jax.experimental.pallas import tpu as pltpu

_MAXP = 128
_NPOS = 2 * _MAXP + 1
_D = 32


def _make_body(S, BQ, grid):
    EDGE = S - 1 - _MAXP

    def body(table_ref, out_ref, big_ref, sem):
        i = pl.program_id(0)

        @pl.when(i == 0)
        def _build_big():
            big_ref[0:EDGE, :] = jnp.broadcast_to(table_ref[0:1, :], (EDGE, _D))
            big_ref[EDGE:EDGE + _NPOS, :] = table_ref[:, :]
            big_ref[EDGE + _NPOS:2 * S - 1, :] = jnp.broadcast_to(
                table_ref[_NPOS - 1:_NPOS, :], (EDGE, _D))

        for k in range(BQ):
            q = i * BQ + k
            off = S - 1 - q
            pltpu.make_async_copy(
                big_ref.at[pl.ds(off, S)], out_ref.at[q], sem).start()

        @pl.when(i > 0)
        def _wait_prev():
            for _ in range(BQ):
                pltpu.make_async_copy(
                    big_ref.at[pl.ds(0, S)], out_ref.at[0], sem).wait()

        @pl.when(i == grid - 1)
        def _drain():
            for _ in range(BQ):
                pltpu.make_async_copy(
                    big_ref.at[pl.ds(0, S)], out_ref.at[0], sem).wait()

    return body


def kernel(inputs, table):
    S = inputs.shape[1]
    BQ = 16
    grid = S // BQ
    return pl.pallas_call(
        _make_body(S, BQ, grid),
        grid=(grid,),
        in_specs=[pl.BlockSpec(memory_space=pltpu.MemorySpace.VMEM)],
        out_specs=pl.BlockSpec(memory_space=pl.ANY),
        out_shape=jax.ShapeDtypeStruct((S, S, _D), jnp.float32),
        scratch_shapes=[
            pltpu.VMEM((2 * S - 1, _D), jnp.float32),
            pltpu.SemaphoreType.DMA,
        ],
    )(table)


# speedup vs baseline: 16.2401x; 1.8166x over previous
import jax
import jax.numpy as jnp
from jax.experimental import pallas as pl
from jax.experimental.pallas import tpu as pltpu

_MAXP = 128
_NPOS = 2 * _MAXP + 1
_D = 32


def _make_body(S, BQ, grid):
    W = S * _D // 128
    U = (2 * S - 1) // 4 + 1

    def body(table_ref, out_ref, g_ref, sem):
        i = pl.program_id(0)

        @pl.when(i == 0)
        def _build_g():
            t = table_ref[:, :]
            t0 = t[0:1, :]
            t_last = t[_NPOS - 1:_NPOS, :]

            def clamped(lo_pad, hi_pad):
                return jnp.concatenate(
                    [jnp.broadcast_to(t0, (lo_pad, _D)), t,
                     jnp.broadcast_to(t_last, (hi_pad, _D))], axis=0)

            ts = jnp.concatenate(
                [clamped(4 - cc, 3 + cc) for cc in range(4)], axis=1)
            n_iota = jax.lax.broadcasted_iota(jnp.int32, (U, _NPOS + 7), 1)
            u_iota = jax.lax.broadcasted_iota(jnp.int32, (U, _NPOS + 7), 0)
            for p in range(4):
                n0 = jnp.clip(4 * u_iota + (p - (S - 1 - _MAXP) + 4), 0,
                              _NPOS + 3)
                onehot = (n_iota == n0).astype(jnp.float32)
                g_ref[p, :, :] = jax.lax.dot_general(
                    onehot, ts, (((1,), (0,)), ((), ())),
                    preferred_element_type=jnp.float32)

        for k in range(BQ):
            q = i * BQ + k
            phase = (S - 1 - k) % 4
            e = (S - 1 - q - phase) // 4
            pltpu.make_async_copy(
                g_ref.at[phase, pl.ds(e, W), :], out_ref.at[q], sem).start()

        @pl.when(i > 0)
        def _wait_prev():
            for _ in range(BQ):
                pltpu.make_async_copy(
                    g_ref.at[0, pl.ds(0, W), :], out_ref.at[0], sem).wait()

        @pl.when(i == grid - 1)
        def _drain():
            for _ in range(BQ):
                pltpu.make_async_copy(
                    g_ref.at[0, pl.ds(0, W), :], out_ref.at[0], sem).wait()

    return body


def kernel(inputs, table):
    S = inputs.shape[1]
    BQ = 16
    grid = S // BQ
    W = S * _D // 128
    U = (2 * S - 1) // 4 + 1
    out = pl.pallas_call(
        _make_body(S, BQ, grid),
        grid=(grid,),
        in_specs=[pl.BlockSpec(memory_space=pltpu.MemorySpace.VMEM)],
        out_specs=pl.BlockSpec(memory_space=pl.ANY),
        out_shape=jax.ShapeDtypeStruct((S, W, 128), jnp.float32),
        scratch_shapes=[
            pltpu.VMEM((4, U, 128), jnp.float32),
            pltpu.SemaphoreType.DMA,
        ],
    )(table)
    return out.reshape(S, S, _D)
